# Initial kernel scaffold; baseline (speedup 1.0000x reference)
import jax
import jax.numpy as jnp
from jax.experimental import pallas as pl


def kernel(x, edge_index, edge_weight, W1, b1, W2, b2):
    raise NotImplementedError("write your pallas kernel here")


# speedup vs baseline: 20.8355x; 20.8355x over previous
import functools

import jax
import jax.numpy as jnp
from jax import lax
from jax.experimental import pallas as pl
from jax.experimental.pallas import tpu as pltpu
from jax.experimental.pallas import tpu_sc as plsc

NC = 2
NS = 16
L = 16
NW = NC * NS
CH = 80


def _mesh():
    return plsc.VectorSubcoreMesh(
        core_axis_name="c", subcore_axis_name="s", num_cores=NC, num_subcores=NS
    )


@functools.lru_cache(maxsize=None)
def _deg_kernel(n, epw):
    @functools.partial(
        pl.kernel,
        out_type=jax.ShapeDtypeStruct((NW, n), jnp.float32),
        mesh=_mesh(),
        scratch_types=[
            pltpu.VMEM((epw,), jnp.int32),
            pltpu.VMEM((epw,), jnp.float32),
            pltpu.VMEM((n,), jnp.float32),
        ],
        compiler_params=pltpu.CompilerParams(use_tc_tiling_on_sc=False, needs_layout_passes=False),
    )
    def deg_k(col_hbm, w_hbm, out_hbm, col_v, w_v, deg_v):
        cid = lax.axis_index("c")
        sid = lax.axis_index("s")
        wid = sid * NC + cid
        pltpu.sync_copy(col_hbm.at[wid], col_v)
        pltpu.sync_copy(w_hbm.at[wid], w_v)

        def zb(i, c):
            deg_v[pl.ds(i * L, L)] = jnp.zeros((L,), jnp.float32)
            return c

        lax.fori_loop(0, n // L, zb, 0)

        def eb(i, c):
            plsc.addupdate_scatter(deg_v, [col_v[pl.ds(i * L, L)]],
                                   w_v[pl.ds(i * L, L)])
            return c

        lax.fori_loop(0, epw // L, eb, 0)
        pltpu.sync_copy(deg_v, out_hbm.at[wid])

    return deg_k


@functools.lru_cache(maxsize=None)
def _prop_kernel(n, d, nch):
    rps = n // NS

    @functools.partial(
        pl.kernel,
        out_type=jax.ShapeDtypeStruct((NC, n, d), jnp.float32),
        mesh=_mesh(),
        scratch_types=[
            pltpu.VMEM((nch, CH), jnp.int32),
            pltpu.VMEM((nch, CH), jnp.int32),
            pltpu.VMEM((nch, CH), jnp.float32),
            pltpu.VMEM((CH, d), jnp.float32),
            pltpu.VMEM_SHARED((n, d), jnp.float32),
            pltpu.SemaphoreType.DMA,
        ],
        compiler_params=pltpu.CompilerParams(use_tc_tiling_on_sc=False, needs_layout_passes=False),
    )
    def prop_k(y_hbm, row_hbm, col_hbm, w_hbm, zero_hbm, out_hbm,
               row_v, col_v, w_v, gbuf, acc_sh, sem):
        cid = lax.axis_index("c")
        sid = lax.axis_index("s")
        wid = sid * NC + cid

        pltpu.sync_copy(zero_hbm.at[pl.ds(sid * rps, rps)],
                        acc_sh.at[pl.ds(sid * rps, rps)])
        pltpu.sync_copy(row_hbm.at[wid], row_v)
        pltpu.sync_copy(col_hbm.at[wid], col_v)
        pltpu.sync_copy(w_hbm.at[wid], w_v)
        plsc.subcore_barrier()

        def chunk(j, c):
            pltpu.async_copy(y_hbm.at[row_v.at[j]], gbuf, sem).wait()
            jv = jnp.full((L,), j, jnp.int32)

            def rowb(r, c2):
                wspl = plsc.load_gather(w_v, [jv, jnp.full((L,), r, jnp.int32)])
                for k in range(d // L):
                    gbuf[r, pl.ds(k * L, L)] = gbuf[r, pl.ds(k * L, L)] * wspl
                return c2

            lax.fori_loop(0, CH, rowb, 0)
            pltpu.sync_copy(gbuf, acc_sh.at[col_v.at[j]], add=True)
            return c

        lax.fori_loop(0, nch, chunk, 0)
        plsc.subcore_barrier()
        pltpu.sync_copy(acc_sh.at[pl.ds(sid * rps, rps)],
                        out_hbm.at[cid, pl.ds(sid * rps, rps)])

    return prop_k


def _tc1_body(degp_ref, x_ref, w1_ref, y1_ref, dis_ref):
    deg = lax.dot_general(degp_ref[...], jnp.ones((NW, 1), jnp.float32),
                          (((0,), (0,)), ((), ())),
                          preferred_element_type=jnp.float32) + 1.0
    dis = lax.rsqrt(deg)
    xw = jnp.dot(x_ref[...], w1_ref[...], preferred_element_type=jnp.float32)
    y1_ref[...] = dis * xw
    dis_ref[...] = dis


def _tc2_body(acc_ref, y1_ref, dis_ref, b1_ref, w2_ref, y2_ref):
    dis = dis_ref[...]
    s = dis * (acc_ref[0] + acc_ref[1] + y1_ref[...]) + b1_ref[...]
    h = jnp.maximum(s, 0.0)
    y2_ref[...] = dis * jnp.dot(h, w2_ref[...], preferred_element_type=jnp.float32)


def _tc3_body(acc_ref, y2_ref, dis_ref, b2_ref, o_ref):
    o = dis_ref[...] * (acc_ref[0] + acc_ref[1] + y2_ref[...]) + b2_ref[...]
    m = jnp.max(o, axis=1, keepdims=True)
    s = o - m
    o_ref[...] = s - jnp.log(jnp.sum(jnp.exp(s), axis=1, keepdims=True))


def kernel(x, edge_index, edge_weight, W1, b1, W2, b2):
    n, d_in = x.shape
    d_hid = W1.shape[1]
    d_out = W2.shape[1]
    e = edge_weight.shape[0]
    assert e % (NW * CH) == 0 and n % NS == 0
    nch = e // (NW * CH)

    f32 = jnp.float32
    row3 = edge_index[0].astype(jnp.int32).reshape(NW, nch, CH)
    col3 = edge_index[1].astype(jnp.int32).reshape(NW, nch, CH)
    w3 = edge_weight.astype(f32).reshape(NW, nch, CH)
    epw = e // NW
    zh = jnp.zeros((n, d_hid), f32)
    zo = jnp.zeros((n, d_out), f32)

    degp = _deg_kernel(n, epw)(col3.reshape(NW, epw), w3.reshape(NW, epw))

    y1, dis = pl.pallas_call(
        _tc1_body,
        out_shape=(
            jax.ShapeDtypeStruct((n, d_hid), f32),
            jax.ShapeDtypeStruct((n, 1), f32),
        ),
    )(degp, x, W1)

    acc1 = _prop_kernel(n, d_hid, nch)(y1, row3, col3, w3, zh)

    y2 = pl.pallas_call(
        _tc2_body,
        out_shape=jax.ShapeDtypeStruct((n, d_out), f32),
    )(acc1, y1, dis, b1.reshape(1, d_hid), W2)

    acc2 = _prop_kernel(n, d_out, nch)(y2, row3, col3, w3, zo)

    out = pl.pallas_call(
        _tc3_body,
        out_shape=jax.ShapeDtypeStruct((n, d_out), f32),
    )(acc2, y2, dis, b2.reshape(1, d_out))

    return out
